# baseline (speedup 1.0000x reference)
import functools

import jax
import jax.numpy as jnp
from jax import lax
from jax.experimental import pallas as pl
from jax.experimental.pallas import tpu as pltpu
from jax.experimental.pallas import tpu_sc as plsc

_N = 10000
_E = 320000
_H = 128
_DO = 64

_NC = 2
_NS = 16
_NW = _NC * _NS
_CHUNK = 128
_K = 80
_KROWS = _NW * _K
_EPAD = _KROWS * _CHUNK
_NACC = 10112
_ZR = _NACC // _NS
_G = 40

_BR = 1000


def _sc_mesh():
    return plsc.VectorSubcoreMesh(core_axis_name="c", subcore_axis_name="s")


def _sc_degree(dstp, ones, z128):
    @functools.partial(
        pl.kernel,
        out_type=jax.ShapeDtypeStruct((_NC, _NACC, _H), jnp.float32),
        mesh=_sc_mesh(),
        scratch_types=[
            pltpu.VMEM((_K, _CHUNK), jnp.int32),
            pltpu.VMEM((_CHUNK, _H), jnp.float32),
            pltpu.VMEM_SHARED((_NACC, _H), jnp.float32),
        ],
    )
    def k(dst_hbm, ones_hbm, z_hbm, out_hbm, idx_v, ones_v, acc_sh):
        c = lax.axis_index("c")
        s = lax.axis_index("s")
        w = c * _NS + s
        pltpu.sync_copy(z_hbm.at[pl.ds(s * _ZR, _ZR)],
                        acc_sh.at[pl.ds(s * _ZR, _ZR)])
        pltpu.sync_copy(ones_hbm, ones_v)
        pltpu.sync_copy(dst_hbm.at[pl.ds(w * _K, _K)], idx_v)
        plsc.subcore_barrier()

        @pl.loop(0, _K)
        def _(j):
            pltpu.sync_copy(ones_v, acc_sh.at[idx_v.at[j]], add=True)

        plsc.subcore_barrier()
        pltpu.sync_copy(acc_sh.at[pl.ds(s * _ZR, _ZR)],
                        out_hbm.at[c, pl.ds(s * _ZR, _ZR)])

    return k(dstp, ones, z128)


def _sc_scatter(g, srcp, dstp, zD, D):
    @functools.partial(
        pl.kernel,
        out_type=jax.ShapeDtypeStruct((_NC, _NACC, D), jnp.float32),
        mesh=_sc_mesh(),
        scratch_types=[
            pltpu.VMEM((_G, _CHUNK), jnp.int32),
            pltpu.VMEM((_G, _CHUNK), jnp.int32),
            pltpu.VMEM((2, _CHUNK, D), jnp.float32),
            pltpu.VMEM_SHARED((_NACC, D), jnp.float32),
            pltpu.SemaphoreType.DMA,
            pltpu.SemaphoreType.DMA,
            pltpu.SemaphoreType.DMA,
            pltpu.SemaphoreType.DMA,
        ],
    )
    def k(g_hbm, src_hbm, dst_hbm, z_hbm, out_hbm, src_v, dst_v, rows_v,
          acc_sh, semg0, semg1, sems0, sems1):
        c = lax.axis_index("c")
        s = lax.axis_index("s")
        w = c * _NS + s
        pltpu.sync_copy(z_hbm.at[pl.ds(s * _ZR, _ZR)],
                        acc_sh.at[pl.ds(s * _ZR, _ZR)])
        plsc.subcore_barrier()

        r0 = rows_v.at[0]
        r1 = rows_v.at[1]

        @pl.loop(0, _K, step=_G)
        def _(q):
            pltpu.sync_copy(src_hbm.at[pl.ds(w * _K + q, _G)], src_v)
            pltpu.sync_copy(dst_hbm.at[pl.ds(w * _K + q, _G)], dst_v)
            pltpu.async_copy(g_hbm.at[src_v.at[0]], r0, semg0)
            pltpu.async_copy(g_hbm.at[src_v.at[1]], r1, semg1)

            @pl.loop(0, _G - 2, step=2)
            def _(j):
                pltpu.make_async_copy(g_hbm.at[src_v.at[j]], r0, semg0).wait()
                pltpu.async_copy(r0, acc_sh.at[dst_v.at[j]], sems0, add=True)
                pltpu.make_async_copy(
                    g_hbm.at[src_v.at[j + 1]], r1, semg1).wait()
                pltpu.async_copy(r1, acc_sh.at[dst_v.at[j + 1]], sems1,
                                 add=True)
                pltpu.make_async_copy(
                    r0, acc_sh.at[dst_v.at[j]], sems0).wait()
                pltpu.async_copy(g_hbm.at[src_v.at[j + 2]], r0, semg0)
                pltpu.make_async_copy(
                    r1, acc_sh.at[dst_v.at[j + 1]], sems1).wait()
                pltpu.async_copy(g_hbm.at[src_v.at[j + 3]], r1, semg1)

            pltpu.make_async_copy(g_hbm.at[src_v.at[_G - 2]], r0, semg0).wait()
            pltpu.async_copy(r0, acc_sh.at[dst_v.at[_G - 2]], sems0, add=True)
            pltpu.make_async_copy(g_hbm.at[src_v.at[_G - 1]], r1, semg1).wait()
            pltpu.async_copy(r1, acc_sh.at[dst_v.at[_G - 1]], sems1, add=True)
            pltpu.make_async_copy(
                r0, acc_sh.at[dst_v.at[_G - 2]], sems0).wait()
            pltpu.make_async_copy(
                r1, acc_sh.at[dst_v.at[_G - 1]], sems1).wait()

        plsc.subcore_barrier()
        pltpu.sync_copy(acc_sh.at[pl.ds(s * _ZR, _ZR)],
                        out_hbm.at[c, pl.ds(s * _ZR, _ZR)])

    return k(g, srcp, dstp, zD)


def _tc_prep(degp):
    def body(p_ref, d_ref):
        deg = p_ref[0, :, :1] + p_ref[1, :, :1] + 1.0
        d_ref[...] = lax.rsqrt(deg[:_N, :])

    return pl.pallas_call(
        body,
        out_shape=jax.ShapeDtypeStruct((_N, 1), jnp.float32),
        in_specs=[pl.BlockSpec((_NC, _NACC, _H), lambda: (0, 0, 0))],
        out_specs=pl.BlockSpec((_N, 1), lambda: (0, 0)),
    )(degp)


def _tc_g1(x, W1, d):
    def body(x_ref, w_ref, d_ref, o_ref):
        h = jnp.dot(x_ref[...], w_ref[...], preferred_element_type=jnp.float32,
                    precision=lax.Precision.HIGHEST)
        o_ref[...] = h * d_ref[...]

    return pl.pallas_call(
        body,
        grid=(_N // _BR,),
        out_shape=jax.ShapeDtypeStruct((_N, _H), jnp.float32),
        in_specs=[
            pl.BlockSpec((_BR, _H), lambda i: (i, 0)),
            pl.BlockSpec((_H, _H), lambda i: (0, 0)),
            pl.BlockSpec((_BR, 1), lambda i: (i, 0)),
        ],
        out_specs=pl.BlockSpec((_BR, _H), lambda i: (i, 0)),
    )(x, W1, d)


def _tc_layer(p, g, d, b, W, dout):
    din = g.shape[1]

    def body(p_ref, g_ref, d_ref, b_ref, w_ref, o_ref):
        ssum = p_ref[0] + p_ref[1] + g_ref[...]
        z = jnp.maximum(d_ref[...] * ssum + b_ref[...], 0.0)
        o_ref[...] = jnp.dot(z, w_ref[...], preferred_element_type=jnp.float32,
                             precision=lax.Precision.HIGHEST) * d_ref[...]

    return pl.pallas_call(
        body,
        grid=(_N // _BR,),
        out_shape=jax.ShapeDtypeStruct((_N, dout), jnp.float32),
        in_specs=[
            pl.BlockSpec((_NC, _BR, din), lambda i: (0, i, 0)),
            pl.BlockSpec((_BR, din), lambda i: (i, 0)),
            pl.BlockSpec((_BR, 1), lambda i: (i, 0)),
            pl.BlockSpec((1, din), lambda i: (0, 0)),
            pl.BlockSpec((din, dout), lambda i: (0, 0)),
        ],
        out_specs=pl.BlockSpec((_BR, dout), lambda i: (i, 0)),
    )(p, g, d, b, W)


def _tc_final(p, g, d, b):
    def body(p_ref, g_ref, d_ref, b_ref, o_ref):
        e = d_ref[...] * (p_ref[0] + p_ref[1] + g_ref[...]) + b_ref[...]
        n2 = jnp.sum(e * e, axis=1, keepdims=True)
        o_ref[...] = (e * lax.rsqrt(jnp.maximum(n2, 1e-24)))[:, :_DO]

    return pl.pallas_call(
        body,
        grid=(_N // _BR,),
        out_shape=jax.ShapeDtypeStruct((_N, _DO), jnp.float32),
        in_specs=[
            pl.BlockSpec((_NC, _BR, _H), lambda i: (0, i, 0)),
            pl.BlockSpec((_BR, _H), lambda i: (i, 0)),
            pl.BlockSpec((_BR, 1), lambda i: (i, 0)),
            pl.BlockSpec((1, _H), lambda i: (0, 0)),
        ],
        out_specs=pl.BlockSpec((_BR, _DO), lambda i: (i, 0)),
    )(p, g, d, b)


def kernel(x, edge_index, W1, b1, W2, b2, W3, b3):
    src = edge_index[0]
    dst = edge_index[1]
    pad = _EPAD - _E
    pad_iota = jnp.arange(pad, dtype=jnp.int32)
    srcp = jnp.concatenate(
        [src, pad_iota % _N]).reshape(_KROWS, _CHUNK)
    dstp = jnp.concatenate(
        [dst, _N + pad_iota % (_NACC - _N)]).reshape(_KROWS, _CHUNK)
    ones128 = jnp.ones((_CHUNK, _H), jnp.float32)
    z128 = jnp.zeros((_NACC, _H), jnp.float32)
    W3p = jnp.concatenate([W3, jnp.zeros((_H, _H - _DO), jnp.float32)], axis=1)
    b3p = jnp.concatenate([b3, jnp.zeros((_H - _DO,), jnp.float32)])

    degp = _sc_degree(dstp, ones128, z128)
    d = _tc_prep(degp)

    g1 = _tc_g1(x, W1, d)
    p1 = _sc_scatter(g1, srcp, dstp, z128, _H)
    g2 = _tc_layer(p1, g1, d, b1.reshape(1, _H), W2, _H)
    p2 = _sc_scatter(g2, srcp, dstp, z128, _H)
    g3 = _tc_layer(p2, g2, d, b2.reshape(1, _H), W3p, _H)
    p3 = _sc_scatter(g3, srcp, dstp, z128, _H)
    return _tc_final(p3, g3, d, b3p.reshape(1, _H))


# speedup vs baseline: 1.2137x; 1.2137x over previous
import functools

import jax
import jax.numpy as jnp
from jax import lax
from jax.experimental import pallas as pl
from jax.experimental.pallas import tpu as pltpu
from jax.experimental.pallas import tpu_sc as plsc

_N = 10000
_E = 320000
_H = 128
_DO = 64

_NC = 2
_NS = 16
_NW = _NC * _NS
_CHUNK = 128
_K = 80
_KROWS = _NW * _K
_EPAD = _KROWS * _CHUNK
_NACC = 10112
_ZR = _NACC // _NS
_G = 40

_BR = 1000


def _sc_mesh():
    return plsc.VectorSubcoreMesh(core_axis_name="c", subcore_axis_name="s")


def _sc_degree(dstp, ones, z128):
    @functools.partial(
        pl.kernel,
        out_type=jax.ShapeDtypeStruct((_NC, _NACC, _H), jnp.float32),
        mesh=_sc_mesh(),
        scratch_types=[
            pltpu.VMEM((_K, _CHUNK), jnp.int32),
            pltpu.VMEM((_CHUNK, _H), jnp.float32),
            pltpu.VMEM_SHARED((_NACC, _H), jnp.float32),
        ],
    )
    def k(dst_hbm, ones_hbm, z_hbm, out_hbm, idx_v, ones_v, acc_sh):
        c = lax.axis_index("c")
        s = lax.axis_index("s")
        w = c * _NS + s
        pltpu.sync_copy(z_hbm.at[pl.ds(s * _ZR, _ZR)],
                        acc_sh.at[pl.ds(s * _ZR, _ZR)])
        pltpu.sync_copy(ones_hbm, ones_v)
        pltpu.sync_copy(dst_hbm.at[pl.ds(w * _K, _K)], idx_v)
        plsc.subcore_barrier()

        @pl.loop(0, _K)
        def _(j):
            pltpu.sync_copy(ones_v, acc_sh.at[idx_v.at[j]], add=True)

        plsc.subcore_barrier()
        pltpu.sync_copy(acc_sh.at[pl.ds(s * _ZR, _ZR)],
                        out_hbm.at[c, pl.ds(s * _ZR, _ZR)])

    return k(dstp, ones, z128)


def _sc_scatter(g, srcp, dstp, zD, D):
    @functools.partial(
        pl.kernel,
        out_type=jax.ShapeDtypeStruct((_NC, _NACC, D), jnp.float32),
        mesh=_sc_mesh(),
        scratch_types=[
            pltpu.VMEM((_G, _CHUNK), jnp.int32),
            pltpu.VMEM((_G, _CHUNK), jnp.int32),
            pltpu.VMEM((2, _CHUNK, D), jnp.float32),
            pltpu.VMEM_SHARED((_NACC, D), jnp.float32),
            pltpu.SemaphoreType.DMA,
            pltpu.SemaphoreType.DMA,
        ],
    )
    def k(g_hbm, src_hbm, dst_hbm, z_hbm, out_hbm, src_v, dst_v, rows_v,
          acc_sh, semg0, semg1):
        c = lax.axis_index("c")
        s = lax.axis_index("s")
        w = c * _NS + s
        pltpu.sync_copy(z_hbm.at[pl.ds(s * _ZR, _ZR)],
                        acc_sh.at[pl.ds(s * _ZR, _ZR)])
        plsc.subcore_barrier()

        r0 = rows_v.at[0]
        r1 = rows_v.at[1]

        @pl.loop(0, _K, step=_G)
        def _(q):
            pltpu.sync_copy(src_hbm.at[pl.ds(w * _K + q, _G)], src_v)
            pltpu.sync_copy(dst_hbm.at[pl.ds(w * _K + q, _G)], dst_v)
            pltpu.async_copy(g_hbm.at[src_v.at[0]], r0, semg0)

            @pl.loop(0, _G, step=2)
            def _(j):
                pltpu.async_copy(g_hbm.at[src_v.at[j + 1]], r1, semg1)
                pltpu.make_async_copy(g_hbm.at[src_v.at[j]], r0, semg0).wait()
                pltpu.sync_copy(r0, acc_sh.at[dst_v.at[j]], add=True)

                @pl.when(j + 2 < _G)
                def _():
                    pltpu.async_copy(g_hbm.at[src_v.at[j + 2]], r0, semg0)

                pltpu.make_async_copy(
                    g_hbm.at[src_v.at[j + 1]], r1, semg1).wait()
                pltpu.sync_copy(r1, acc_sh.at[dst_v.at[j + 1]], add=True)

        plsc.subcore_barrier()
        pltpu.sync_copy(acc_sh.at[pl.ds(s * _ZR, _ZR)],
                        out_hbm.at[c, pl.ds(s * _ZR, _ZR)])

    return k(g, srcp, dstp, zD)


def _tc_prep(degp):
    def body(p_ref, d_ref):
        deg = p_ref[0, :, :1] + p_ref[1, :, :1] + 1.0
        d_ref[...] = lax.rsqrt(deg[:_N, :])

    return pl.pallas_call(
        body,
        out_shape=jax.ShapeDtypeStruct((_N, 1), jnp.float32),
        in_specs=[pl.BlockSpec((_NC, _NACC, _H), lambda: (0, 0, 0))],
        out_specs=pl.BlockSpec((_N, 1), lambda: (0, 0)),
    )(degp)


def _tc_g1(x, W1, d):
    def body(x_ref, w_ref, d_ref, o_ref):
        h = jnp.dot(x_ref[...], w_ref[...], preferred_element_type=jnp.float32,
                    precision=lax.Precision.HIGHEST)
        o_ref[...] = h * d_ref[...]

    return pl.pallas_call(
        body,
        grid=(_N // _BR,),
        out_shape=jax.ShapeDtypeStruct((_N, _H), jnp.float32),
        in_specs=[
            pl.BlockSpec((_BR, _H), lambda i: (i, 0)),
            pl.BlockSpec((_H, _H), lambda i: (0, 0)),
            pl.BlockSpec((_BR, 1), lambda i: (i, 0)),
        ],
        out_specs=pl.BlockSpec((_BR, _H), lambda i: (i, 0)),
    )(x, W1, d)


def _tc_layer(p, g, d, b, W, dout):
    din = g.shape[1]

    def body(p_ref, g_ref, d_ref, b_ref, w_ref, o_ref):
        ssum = p_ref[0] + p_ref[1] + g_ref[...]
        z = jnp.maximum(d_ref[...] * ssum + b_ref[...], 0.0)
        o_ref[...] = jnp.dot(z, w_ref[...], preferred_element_type=jnp.float32,
                             precision=lax.Precision.HIGHEST) * d_ref[...]

    return pl.pallas_call(
        body,
        grid=(_N // _BR,),
        out_shape=jax.ShapeDtypeStruct((_N, dout), jnp.float32),
        in_specs=[
            pl.BlockSpec((_NC, _BR, din), lambda i: (0, i, 0)),
            pl.BlockSpec((_BR, din), lambda i: (i, 0)),
            pl.BlockSpec((_BR, 1), lambda i: (i, 0)),
            pl.BlockSpec((1, din), lambda i: (0, 0)),
            pl.BlockSpec((din, dout), lambda i: (0, 0)),
        ],
        out_specs=pl.BlockSpec((_BR, dout), lambda i: (i, 0)),
    )(p, g, d, b, W)


def _tc_final(p, g, d, b):
    def body(p_ref, g_ref, d_ref, b_ref, o_ref):
        e = d_ref[...] * (p_ref[0] + p_ref[1] + g_ref[...]) + b_ref[...]
        n2 = jnp.sum(e * e, axis=1, keepdims=True)
        o_ref[...] = (e * lax.rsqrt(jnp.maximum(n2, 1e-24)))[:, :_DO]

    return pl.pallas_call(
        body,
        grid=(_N // _BR,),
        out_shape=jax.ShapeDtypeStruct((_N, _DO), jnp.float32),
        in_specs=[
            pl.BlockSpec((_NC, _BR, _H), lambda i: (0, i, 0)),
            pl.BlockSpec((_BR, _H), lambda i: (i, 0)),
            pl.BlockSpec((_BR, 1), lambda i: (i, 0)),
            pl.BlockSpec((1, _H), lambda i: (0, 0)),
        ],
        out_specs=pl.BlockSpec((_BR, _DO), lambda i: (i, 0)),
    )(p, g, d, b)


def kernel(x, edge_index, W1, b1, W2, b2, W3, b3):
    src = edge_index[0]
    dst = edge_index[1]
    pad = _EPAD - _E
    pad_iota = jnp.arange(pad, dtype=jnp.int32)
    srcp = jnp.concatenate(
        [src, pad_iota % _N]).reshape(_KROWS, _CHUNK)
    dstp = jnp.concatenate(
        [dst, _N + pad_iota % (_NACC - _N)]).reshape(_KROWS, _CHUNK)
    ones128 = jnp.ones((_CHUNK, _H), jnp.float32)
    z128 = jnp.zeros((_NACC, _H), jnp.float32)
    W3p = jnp.concatenate([W3, jnp.zeros((_H, _H - _DO), jnp.float32)], axis=1)
    b3p = jnp.concatenate([b3, jnp.zeros((_H - _DO,), jnp.float32)])

    degp = _sc_degree(dstp, ones128, z128)
    d = _tc_prep(degp)

    g1 = _tc_g1(x, W1, d)
    p1 = _sc_scatter(g1, srcp, dstp, z128, _H)
    g2 = _tc_layer(p1, g1, d, b1.reshape(1, _H), W2, _H)
    p2 = _sc_scatter(g2, srcp, dstp, z128, _H)
    g3 = _tc_layer(p2, g2, d, b2.reshape(1, _H), W3p, _H)
    p3 = _sc_scatter(g3, srcp, dstp, z128, _H)
    return _tc_final(p3, g3, d, b3p.reshape(1, _H))


# speedup vs baseline: 1.2596x; 1.0379x over previous
import functools

import jax
import jax.numpy as jnp
from jax import lax
from jax.experimental import pallas as pl
from jax.experimental.pallas import tpu as pltpu
from jax.experimental.pallas import tpu_sc as plsc

_N = 10000
_E = 320000
_H = 128
_DO = 64

_NC = 2
_NS = 16
_NW = _NC * _NS
_CHUNK = 128
_K = 80
_KROWS = _NW * _K
_EPAD = _KROWS * _CHUNK
_NACC = 10112
_ZR = _NACC // _NS
_G = 40

_BR = 2000


def _sc_mesh():
    return plsc.VectorSubcoreMesh(core_axis_name="c", subcore_axis_name="s")


def _sc_degree(dstp, ones, z128):
    @functools.partial(
        pl.kernel,
        out_type=jax.ShapeDtypeStruct((_NC, _NACC, _H), jnp.float32),
        mesh=_sc_mesh(),
        scratch_types=[
            pltpu.VMEM((_K, _CHUNK), jnp.int32),
            pltpu.VMEM((_CHUNK, _H), jnp.float32),
            pltpu.VMEM_SHARED((_NACC, _H), jnp.float32),
        ],
    )
    def k(dst_hbm, ones_hbm, z_hbm, out_hbm, idx_v, ones_v, acc_sh):
        c = lax.axis_index("c")
        s = lax.axis_index("s")
        w = c * _NS + s
        pltpu.sync_copy(z_hbm.at[pl.ds(s * _ZR, _ZR)],
                        acc_sh.at[pl.ds(s * _ZR, _ZR)])
        pltpu.sync_copy(ones_hbm, ones_v)
        pltpu.sync_copy(dst_hbm.at[pl.ds(w * _K, _K)], idx_v)
        plsc.subcore_barrier()

        @pl.loop(0, _K)
        def _(j):
            pltpu.sync_copy(ones_v, acc_sh.at[idx_v.at[j]], add=True)

        plsc.subcore_barrier()
        pltpu.sync_copy(acc_sh.at[pl.ds(s * _ZR, _ZR)],
                        out_hbm.at[c, pl.ds(s * _ZR, _ZR)])

    return k(dstp, ones, z128)


def _sc_scatter(g, srcp, dstp, zD, D):
    @functools.partial(
        pl.kernel,
        out_type=jax.ShapeDtypeStruct((_NC, _NACC, D), jnp.float32),
        mesh=_sc_mesh(),
        scratch_types=[
            pltpu.VMEM((_G, _CHUNK), jnp.int32),
            pltpu.VMEM((_G, _CHUNK), jnp.int32),
            pltpu.VMEM((2, _CHUNK, D), jnp.float32),
            pltpu.VMEM_SHARED((_NACC, D), jnp.float32),
            pltpu.SemaphoreType.DMA,
            pltpu.SemaphoreType.DMA,
        ],
    )
    def k(g_hbm, src_hbm, dst_hbm, z_hbm, out_hbm, src_v, dst_v, rows_v,
          acc_sh, semg0, semg1):
        c = lax.axis_index("c")
        s = lax.axis_index("s")
        w = c * _NS + s
        pltpu.sync_copy(z_hbm.at[pl.ds(s * _ZR, _ZR)],
                        acc_sh.at[pl.ds(s * _ZR, _ZR)])
        plsc.subcore_barrier()

        r0 = rows_v.at[0]
        r1 = rows_v.at[1]

        @pl.loop(0, _K, step=_G)
        def _(q):
            pltpu.sync_copy(src_hbm.at[pl.ds(w * _K + q, _G)], src_v)
            pltpu.sync_copy(dst_hbm.at[pl.ds(w * _K + q, _G)], dst_v)
            pltpu.async_copy(g_hbm.at[src_v.at[0]], r0, semg0)

            @pl.loop(0, _G, step=2)
            def _(j):
                pltpu.async_copy(g_hbm.at[src_v.at[j + 1]], r1, semg1)
                pltpu.make_async_copy(g_hbm.at[src_v.at[j]], r0, semg0).wait()
                pltpu.sync_copy(r0, acc_sh.at[dst_v.at[j]], add=True)

                @pl.when(j + 2 < _G)
                def _():
                    pltpu.async_copy(g_hbm.at[src_v.at[j + 2]], r0, semg0)

                pltpu.make_async_copy(
                    g_hbm.at[src_v.at[j + 1]], r1, semg1).wait()
                pltpu.sync_copy(r1, acc_sh.at[dst_v.at[j + 1]], add=True)

        plsc.subcore_barrier()
        pltpu.sync_copy(acc_sh.at[pl.ds(s * _ZR, _ZR)],
                        out_hbm.at[c, pl.ds(s * _ZR, _ZR)])

    return k(g, srcp, dstp, zD)


def _tc_h1(x, W1):
    def body(x_ref, w_ref, o_ref):
        o_ref[...] = jnp.dot(x_ref[...], w_ref[...],
                             preferred_element_type=jnp.float32,
                             precision=lax.Precision.HIGHEST)

    return pl.pallas_call(
        body,
        grid=(_N // _BR,),
        out_shape=jax.ShapeDtypeStruct((_N, _H), jnp.float32),
        in_specs=[
            pl.BlockSpec((_BR, _H), lambda i: (i, 0)),
            pl.BlockSpec((_H, _H), lambda i: (0, 0)),
        ],
        out_specs=pl.BlockSpec((_BR, _H), lambda i: (i, 0)),
    )(x, W1)


def _tc_prep(degp, h1):
    def body(p_ref, h_ref, d_ref, g_ref):
        i = pl.program_id(0)
        deg = p_ref[0, :, :1] + p_ref[1, :, :1] + 1.0
        d = lax.rsqrt(deg[:_BR, :])
        d_ref[...] = d
        g_ref[...] = h_ref[...] * d

    return pl.pallas_call(
        body,
        grid=(_N // _BR,),
        out_shape=[
            jax.ShapeDtypeStruct((_N, 1), jnp.float32),
            jax.ShapeDtypeStruct((_N, _H), jnp.float32),
        ],
        in_specs=[
            pl.BlockSpec((_NC, _BR, _H), lambda i: (0, i, 0)),
            pl.BlockSpec((_BR, _H), lambda i: (i, 0)),
        ],
        out_specs=[
            pl.BlockSpec((_BR, 1), lambda i: (i, 0)),
            pl.BlockSpec((_BR, _H), lambda i: (i, 0)),
        ],
    )(degp, h1)


def _tc_layer(p, g, d, b, W, dout):
    din = g.shape[1]

    def body(p_ref, g_ref, d_ref, b_ref, w_ref, o_ref):
        ssum = p_ref[0] + p_ref[1] + g_ref[...]
        z = jnp.maximum(d_ref[...] * ssum + b_ref[...], 0.0)
        o_ref[...] = jnp.dot(z, w_ref[...], preferred_element_type=jnp.float32,
                             precision=lax.Precision.HIGHEST) * d_ref[...]

    return pl.pallas_call(
        body,
        grid=(_N // _BR,),
        out_shape=jax.ShapeDtypeStruct((_N, dout), jnp.float32),
        in_specs=[
            pl.BlockSpec((_NC, _BR, din), lambda i: (0, i, 0)),
            pl.BlockSpec((_BR, din), lambda i: (i, 0)),
            pl.BlockSpec((_BR, 1), lambda i: (i, 0)),
            pl.BlockSpec((1, din), lambda i: (0, 0)),
            pl.BlockSpec((din, dout), lambda i: (0, 0)),
        ],
        out_specs=pl.BlockSpec((_BR, dout), lambda i: (i, 0)),
    )(p, g, d, b, W)


def _tc_final(p, g, d, b):
    def body(p_ref, g_ref, d_ref, b_ref, o_ref):
        e = d_ref[...] * (p_ref[0] + p_ref[1] + g_ref[...]) + b_ref[...]
        n2 = jnp.sum(e * e, axis=1, keepdims=True)
        o_ref[...] = (e * lax.rsqrt(jnp.maximum(n2, 1e-24)))[:, :_DO]

    return pl.pallas_call(
        body,
        grid=(_N // _BR,),
        out_shape=jax.ShapeDtypeStruct((_N, _DO), jnp.float32),
        in_specs=[
            pl.BlockSpec((_NC, _BR, _H), lambda i: (0, i, 0)),
            pl.BlockSpec((_BR, _H), lambda i: (i, 0)),
            pl.BlockSpec((_BR, 1), lambda i: (i, 0)),
            pl.BlockSpec((1, _H), lambda i: (0, 0)),
        ],
        out_specs=pl.BlockSpec((_BR, _DO), lambda i: (i, 0)),
    )(p, g, d, b)


def kernel(x, edge_index, W1, b1, W2, b2, W3, b3):
    src = edge_index[0]
    dst = edge_index[1]
    pad = _EPAD - _E
    pad_iota = jnp.arange(pad, dtype=jnp.int32)
    srcp = jnp.concatenate(
        [src, pad_iota % _N]).reshape(_KROWS, _CHUNK)
    dstp = jnp.concatenate(
        [dst, _N + pad_iota % (_NACC - _N)]).reshape(_KROWS, _CHUNK)
    ones128 = jnp.ones((_CHUNK, _H), jnp.float32)
    z128 = jnp.zeros((_NACC, _H), jnp.float32)
    W3p = jnp.concatenate([W3, jnp.zeros((_H, _H - _DO), jnp.float32)], axis=1)
    b3p = jnp.concatenate([b3, jnp.zeros((_H - _DO,), jnp.float32)])

    h1 = _tc_h1(x, W1)
    degp = _sc_degree(dstp, ones128, z128)
    d, g1 = _tc_prep(degp, h1)
    p1 = _sc_scatter(g1, srcp, dstp, z128, _H)
    g2 = _tc_layer(p1, g1, d, b1.reshape(1, _H), W2, _H)
    p2 = _sc_scatter(g2, srcp, dstp, z128, _H)
    g3 = _tc_layer(p2, g2, d, b2.reshape(1, _H), W3p, _H)
    p3 = _sc_scatter(g3, srcp, dstp, z128, _H)
    return _tc_final(p3, g3, d, b3p.reshape(1, _H))


# speedup vs baseline: 1.3856x; 1.1000x over previous
import functools

import jax
import jax.numpy as jnp
from jax import lax
from jax.experimental import pallas as pl
from jax.experimental.pallas import tpu as pltpu
from jax.experimental.pallas import tpu_sc as plsc

_N = 10000
_E = 320000
_H = 128
_DO = 64

_NC = 2
_NS = 16
_NW = _NC * _NS
_CHUNK = 128
_K = 80
_KROWS = _NW * _K
_EPAD = _KROWS * _CHUNK
_NACC = 10112
_ZR = _NACC // _NS
_G = 40

_BR = 2000


def _sc_mesh():
    return plsc.VectorSubcoreMesh(core_axis_name="c", subcore_axis_name="s")


def _sc_degree(dstp, ones, z16):
    @functools.partial(
        pl.kernel,
        out_type=jax.ShapeDtypeStruct((_NC, _NACC, 16), jnp.float32),
        mesh=_sc_mesh(),
        scratch_types=[
            pltpu.VMEM((_K, _CHUNK), jnp.int32),
            pltpu.VMEM((_CHUNK, 16), jnp.float32),
            pltpu.VMEM_SHARED((_NACC, 16), jnp.float32),
        ],
        compiler_params=pltpu.CompilerParams(use_tc_tiling_on_sc=False),
    )
    def k(dst_hbm, ones_hbm, z_hbm, out_hbm, idx_v, ones_v, acc_sh):
        c = lax.axis_index("c")
        s = lax.axis_index("s")
        w = c * _NS + s
        pltpu.sync_copy(z_hbm.at[pl.ds(s * _ZR, _ZR)],
                        acc_sh.at[pl.ds(s * _ZR, _ZR)])
        pltpu.sync_copy(ones_hbm, ones_v)
        pltpu.sync_copy(dst_hbm.at[pl.ds(w * _K, _K)], idx_v)
        plsc.subcore_barrier()

        @pl.loop(0, _K)
        def _(j):
            pltpu.sync_copy(ones_v, acc_sh.at[idx_v.at[j]], add=True)

        plsc.subcore_barrier()
        pltpu.sync_copy(acc_sh.at[pl.ds(s * _ZR, _ZR)],
                        out_hbm.at[c, pl.ds(s * _ZR, _ZR)])

    return k(dstp, ones, z16)


def _sc_scatter(g, srcp, dstp, zD, D):
    @functools.partial(
        pl.kernel,
        out_type=jax.ShapeDtypeStruct((_NC, _NACC, D), jnp.float32),
        mesh=_sc_mesh(),
        scratch_types=[
            pltpu.VMEM((_G, _CHUNK), jnp.int32),
            pltpu.VMEM((_G, _CHUNK), jnp.int32),
            pltpu.VMEM((2, _CHUNK, D), jnp.float32),
            pltpu.VMEM_SHARED((_NACC, D), jnp.float32),
            pltpu.SemaphoreType.DMA,
            pltpu.SemaphoreType.DMA,
        ],
    )
    def k(g_hbm, src_hbm, dst_hbm, z_hbm, out_hbm, src_v, dst_v, rows_v,
          acc_sh, semg0, semg1):
        c = lax.axis_index("c")
        s = lax.axis_index("s")
        w = c * _NS + s
        pltpu.sync_copy(z_hbm.at[pl.ds(s * _ZR, _ZR)],
                        acc_sh.at[pl.ds(s * _ZR, _ZR)])
        plsc.subcore_barrier()

        r0 = rows_v.at[0]
        r1 = rows_v.at[1]

        @pl.loop(0, _K, step=_G)
        def _(q):
            pltpu.sync_copy(src_hbm.at[pl.ds(w * _K + q, _G)], src_v)
            pltpu.sync_copy(dst_hbm.at[pl.ds(w * _K + q, _G)], dst_v)
            pltpu.async_copy(g_hbm.at[src_v.at[0]], r0, semg0)

            @pl.loop(0, _G, step=2)
            def _(j):
                pltpu.async_copy(g_hbm.at[src_v.at[j + 1]], r1, semg1)
                pltpu.make_async_copy(g_hbm.at[src_v.at[j]], r0, semg0).wait()
                pltpu.sync_copy(r0, acc_sh.at[dst_v.at[j]], add=True)

                @pl.when(j + 2 < _G)
                def _():
                    pltpu.async_copy(g_hbm.at[src_v.at[j + 2]], r0, semg0)

                pltpu.make_async_copy(
                    g_hbm.at[src_v.at[j + 1]], r1, semg1).wait()
                pltpu.sync_copy(r1, acc_sh.at[dst_v.at[j + 1]], add=True)

        plsc.subcore_barrier()
        pltpu.sync_copy(acc_sh.at[pl.ds(s * _ZR, _ZR)],
                        out_hbm.at[c, pl.ds(s * _ZR, _ZR)])

    return k(g, srcp, dstp, zD)


def _tc_h1(x, W1):
    def body(x_ref, w_ref, o_ref):
        o_ref[...] = jnp.dot(x_ref[...], w_ref[...],
                             preferred_element_type=jnp.float32,
                             precision=lax.Precision.HIGHEST)

    return pl.pallas_call(
        body,
        grid=(_N // _BR,),
        out_shape=jax.ShapeDtypeStruct((_N, _H), jnp.float32),
        in_specs=[
            pl.BlockSpec((_BR, _H), lambda i: (i, 0)),
            pl.BlockSpec((_H, _H), lambda i: (0, 0)),
        ],
        out_specs=pl.BlockSpec((_BR, _H), lambda i: (i, 0)),
    )(x, W1)


def _tc_prep(degp, h1):
    def body(p_ref, h_ref, d_ref, g_ref):
        i = pl.program_id(0)
        deg = p_ref[0, :, :1] + p_ref[1, :, :1] + 1.0
        d = lax.rsqrt(deg[:_BR, :])
        d_ref[...] = d
        g_ref[...] = h_ref[...] * d

    return pl.pallas_call(
        body,
        grid=(_N // _BR,),
        out_shape=[
            jax.ShapeDtypeStruct((_N, 1), jnp.float32),
            jax.ShapeDtypeStruct((_N, _H), jnp.float32),
        ],
        in_specs=[
            pl.BlockSpec((_NC, _BR, 16), lambda i: (0, i, 0)),
            pl.BlockSpec((_BR, _H), lambda i: (i, 0)),
        ],
        out_specs=[
            pl.BlockSpec((_BR, 1), lambda i: (i, 0)),
            pl.BlockSpec((_BR, _H), lambda i: (i, 0)),
        ],
    )(degp, h1)


def _tc_layer(p, g, d, b, W, dout):
    din = g.shape[1]

    def body(p_ref, g_ref, d_ref, b_ref, w_ref, o_ref):
        ssum = p_ref[0] + p_ref[1] + g_ref[...]
        z = jnp.maximum(d_ref[...] * ssum + b_ref[...], 0.0)
        o_ref[...] = jnp.dot(z, w_ref[...], preferred_element_type=jnp.float32,
                             precision=lax.Precision.HIGHEST) * d_ref[...]

    return pl.pallas_call(
        body,
        grid=(_N // _BR,),
        out_shape=jax.ShapeDtypeStruct((_N, dout), jnp.float32),
        in_specs=[
            pl.BlockSpec((_NC, _BR, din), lambda i: (0, i, 0)),
            pl.BlockSpec((_BR, din), lambda i: (i, 0)),
            pl.BlockSpec((_BR, 1), lambda i: (i, 0)),
            pl.BlockSpec((1, din), lambda i: (0, 0)),
            pl.BlockSpec((din, dout), lambda i: (0, 0)),
        ],
        out_specs=pl.BlockSpec((_BR, dout), lambda i: (i, 0)),
    )(p, g, d, b, W)


def _tc_final(p, g, d, b):
    def body(p_ref, g_ref, d_ref, b_ref, o_ref):
        e = d_ref[...] * (p_ref[0] + p_ref[1] + g_ref[...]) + b_ref[...]
        n2 = jnp.sum(e * e, axis=1, keepdims=True)
        o_ref[...] = (e * lax.rsqrt(jnp.maximum(n2, 1e-24)))[:, :_DO]

    return pl.pallas_call(
        body,
        grid=(_N // _BR,),
        out_shape=jax.ShapeDtypeStruct((_N, _DO), jnp.float32),
        in_specs=[
            pl.BlockSpec((_NC, _BR, _H), lambda i: (0, i, 0)),
            pl.BlockSpec((_BR, _H), lambda i: (i, 0)),
            pl.BlockSpec((_BR, 1), lambda i: (i, 0)),
            pl.BlockSpec((1, _H), lambda i: (0, 0)),
        ],
        out_specs=pl.BlockSpec((_BR, _DO), lambda i: (i, 0)),
    )(p, g, d, b)


def kernel(x, edge_index, W1, b1, W2, b2, W3, b3):
    src = edge_index[0]
    dst = edge_index[1]
    pad = _EPAD - _E
    pad_iota = jnp.arange(pad, dtype=jnp.int32)
    srcp = jnp.concatenate(
        [src, pad_iota % _N]).reshape(_KROWS, _CHUNK)
    dstp = jnp.concatenate(
        [dst, _N + pad_iota % (_NACC - _N)]).reshape(_KROWS, _CHUNK)
    ones16 = jnp.ones((_CHUNK, 16), jnp.float32)
    z16 = jnp.zeros((_NACC, 16), jnp.float32)
    z128 = jnp.zeros((_NACC, _H), jnp.float32)
    W3p = jnp.concatenate([W3, jnp.zeros((_H, _H - _DO), jnp.float32)], axis=1)
    b3p = jnp.concatenate([b3, jnp.zeros((_H - _DO,), jnp.float32)])

    h1 = _tc_h1(x, W1)
    degp = _sc_degree(dstp, ones16, z16)
    d, g1 = _tc_prep(degp, h1)
    p1 = _sc_scatter(g1, srcp, dstp, z128, _H)
    g2 = _tc_layer(p1, g1, d, b1.reshape(1, _H), W2, _H)
    p2 = _sc_scatter(g2, srcp, dstp, z128, _H)
    g3 = _tc_layer(p2, g2, d, b2.reshape(1, _H), W3p, _H)
    p3 = _sc_scatter(g3, srcp, dstp, z128, _H)
    return _tc_final(p3, g3, d, b3p.reshape(1, _H))


# speedup vs baseline: 1.3913x; 1.0041x over previous
import functools

import jax
import jax.numpy as jnp
from jax import lax
from jax.experimental import pallas as pl
from jax.experimental.pallas import tpu as pltpu
from jax.experimental.pallas import tpu_sc as plsc

_N = 10000
_E = 320000
_H = 128
_DO = 64

_NC = 2
_NS = 16
_NW = _NC * _NS
_CHUNK = 128
_K = 80
_KROWS = _NW * _K
_EPAD = _KROWS * _CHUNK
_NACC = 10112
_ZR = _NACC // _NS
_G = 40

_BR = 2000


def _sc_mesh():
    return plsc.VectorSubcoreMesh(core_axis_name="c", subcore_axis_name="s")


def _sc_degree(dstp, ones, z16):
    @functools.partial(
        pl.kernel,
        out_type=jax.ShapeDtypeStruct((_NC, _NACC, 16), jnp.float32),
        mesh=_sc_mesh(),
        scratch_types=[
            pltpu.VMEM((_K, _CHUNK), jnp.int32),
            pltpu.VMEM((_CHUNK, 16), jnp.float32),
            pltpu.VMEM_SHARED((_NACC, 16), jnp.float32),
        ],
        compiler_params=pltpu.CompilerParams(use_tc_tiling_on_sc=False),
    )
    def k(dst_hbm, ones_hbm, z_hbm, out_hbm, idx_v, ones_v, acc_sh):
        c = lax.axis_index("c")
        s = lax.axis_index("s")
        w = c * _NS + s
        pltpu.sync_copy(z_hbm.at[pl.ds(s * _ZR, _ZR)],
                        acc_sh.at[pl.ds(s * _ZR, _ZR)])
        pltpu.sync_copy(ones_hbm, ones_v)
        pltpu.sync_copy(dst_hbm.at[pl.ds(w * _K, _K)], idx_v)
        plsc.subcore_barrier()

        @pl.loop(0, _K)
        def _(j):
            pltpu.sync_copy(ones_v, acc_sh.at[idx_v.at[j]], add=True)

        plsc.subcore_barrier()
        pltpu.sync_copy(acc_sh.at[pl.ds(s * _ZR, _ZR)],
                        out_hbm.at[c, pl.ds(s * _ZR, _ZR)])

    return k(dstp, ones, z16)


def _sc_scatter(g, srcp, dstp, zD, D):
    @functools.partial(
        pl.kernel,
        out_type=jax.ShapeDtypeStruct((_NC, _NACC, D), jnp.float32),
        mesh=_sc_mesh(),
        scratch_types=[
            pltpu.VMEM((_G, _CHUNK), jnp.int32),
            pltpu.VMEM((_G, _CHUNK), jnp.int32),
            pltpu.VMEM((2, _CHUNK, D), jnp.float32),
            pltpu.VMEM_SHARED((_NACC, D), jnp.float32),
            pltpu.SemaphoreType.DMA,
            pltpu.SemaphoreType.DMA,
        ],
    )
    def k(g_hbm, src_hbm, dst_hbm, z_hbm, out_hbm, src_v, dst_v, rows_v,
          acc_sh, semg0, semg1):
        c = lax.axis_index("c")
        s = lax.axis_index("s")
        w = c * _NS + s
        pltpu.sync_copy(z_hbm.at[pl.ds(s * _ZR, _ZR)],
                        acc_sh.at[pl.ds(s * _ZR, _ZR)])
        plsc.subcore_barrier()

        r0 = rows_v.at[0]
        r1 = rows_v.at[1]

        @pl.loop(0, _K, step=_G)
        def _(q):
            pltpu.sync_copy(src_hbm.at[pl.ds(w * _K + q, _G)], src_v)
            pltpu.sync_copy(dst_hbm.at[pl.ds(w * _K + q, _G)], dst_v)
            pltpu.async_copy(g_hbm.at[src_v.at[0]], r0, semg0)

            @pl.loop(0, _G, step=2)
            def _(j):
                pltpu.async_copy(g_hbm.at[src_v.at[j + 1]], r1, semg1)
                pltpu.make_async_copy(g_hbm.at[src_v.at[j]], r0, semg0).wait()
                pltpu.sync_copy(r0, acc_sh.at[dst_v.at[j]], add=True)

                @pl.when(j + 2 < _G)
                def _():
                    pltpu.async_copy(g_hbm.at[src_v.at[j + 2]], r0, semg0)

                pltpu.make_async_copy(
                    g_hbm.at[src_v.at[j + 1]], r1, semg1).wait()
                pltpu.sync_copy(r1, acc_sh.at[dst_v.at[j + 1]], add=True)

        plsc.subcore_barrier()
        pltpu.sync_copy(acc_sh.at[pl.ds(s * _ZR, _ZR)],
                        out_hbm.at[c, pl.ds(s * _ZR, _ZR)])

    return k(g, srcp, dstp, zD)


def _tc_h1(x, W1):
    def body(x_ref, w_ref, o_ref):
        o_ref[...] = jnp.dot(x_ref[...], w_ref[...],
                             preferred_element_type=jnp.float32,
                             precision=lax.Precision.HIGHEST)

    return pl.pallas_call(
        body,
        grid=(_N // _BR,),
        out_shape=jax.ShapeDtypeStruct((_N, _H), jnp.float32),
        in_specs=[
            pl.BlockSpec((_BR, _H), lambda i: (i, 0)),
            pl.BlockSpec((_H, _H), lambda i: (0, 0)),
        ],
        out_specs=pl.BlockSpec((_BR, _H), lambda i: (i, 0)),
    )(x, W1)


def _tc_prep(degp, h1):
    def body(p_ref, h_ref, d_ref, g_ref):
        i = pl.program_id(0)
        deg = p_ref[0, :, :1] + p_ref[1, :, :1] + 1.0
        d = lax.rsqrt(deg[:_BR, :])
        d_ref[...] = d
        g_ref[...] = h_ref[...] * d

    return pl.pallas_call(
        body,
        grid=(_N // _BR,),
        out_shape=[
            jax.ShapeDtypeStruct((_N, 1), jnp.float32),
            jax.ShapeDtypeStruct((_N, _H), jnp.float32),
        ],
        in_specs=[
            pl.BlockSpec((_NC, _BR, 16), lambda i: (0, i, 0)),
            pl.BlockSpec((_BR, _H), lambda i: (i, 0)),
        ],
        out_specs=[
            pl.BlockSpec((_BR, 1), lambda i: (i, 0)),
            pl.BlockSpec((_BR, _H), lambda i: (i, 0)),
        ],
    )(degp, h1)


def _tc_layer(p, g, d, b, W, dout):
    din = g.shape[1]

    def body(p_ref, g_ref, d_ref, b_ref, w_ref, o_ref):
        ssum = p_ref[0] + p_ref[1] + g_ref[...]
        z = jnp.maximum(d_ref[...] * ssum + b_ref[...], 0.0)
        o_ref[...] = jnp.dot(z, w_ref[...], preferred_element_type=jnp.float32,
                             precision=lax.Precision.HIGHEST) * d_ref[...]

    return pl.pallas_call(
        body,
        grid=(_N // _BR,),
        out_shape=jax.ShapeDtypeStruct((_N, dout), jnp.float32),
        in_specs=[
            pl.BlockSpec((_NC, _BR, din), lambda i: (0, i, 0)),
            pl.BlockSpec((_BR, din), lambda i: (i, 0)),
            pl.BlockSpec((_BR, 1), lambda i: (i, 0)),
            pl.BlockSpec((1, din), lambda i: (0, 0)),
            pl.BlockSpec((din, dout), lambda i: (0, 0)),
        ],
        out_specs=pl.BlockSpec((_BR, dout), lambda i: (i, 0)),
    )(p, g, d, b, W)


def _tc_final(p, g, d, b):
    def body(p_ref, g_ref, d_ref, b_ref, o_ref):
        e = d_ref[...] * (p_ref[0] + p_ref[1] + g_ref[...]) + b_ref[...]
        n2 = jnp.sum(e * e, axis=1, keepdims=True)
        o_ref[...] = (e * lax.rsqrt(jnp.maximum(n2, 1e-24)))[:, :_DO]

    return pl.pallas_call(
        body,
        grid=(_N // _BR,),
        out_shape=jax.ShapeDtypeStruct((_N, _DO), jnp.float32),
        in_specs=[
            pl.BlockSpec((_NC, _BR, _H), lambda i: (0, i, 0)),
            pl.BlockSpec((_BR, _H), lambda i: (i, 0)),
            pl.BlockSpec((_BR, 1), lambda i: (i, 0)),
            pl.BlockSpec((1, _H), lambda i: (0, 0)),
        ],
        out_specs=pl.BlockSpec((_BR, _DO), lambda i: (i, 0)),
    )(p, g, d, b)


def kernel(x, edge_index, W1, b1, W2, b2, W3, b3):
    src = edge_index[0]
    dst = edge_index[1]
    pad = _EPAD - _E
    pad_iota = jnp.arange(pad, dtype=jnp.int32)
    erows = _E // _CHUNK
    srcp = jnp.concatenate(
        [src.reshape(erows, _CHUNK),
         (pad_iota % _N).reshape(_KROWS - erows, _CHUNK)], axis=0)
    dstp = jnp.concatenate(
        [dst.reshape(erows, _CHUNK),
         (_N + pad_iota % (_NACC - _N)).reshape(_KROWS - erows, _CHUNK)],
        axis=0)
    ones16 = jnp.ones((_CHUNK, 16), jnp.float32)
    z16 = jnp.zeros((_NACC, 16), jnp.float32)
    z128 = jnp.zeros((_NACC, _H), jnp.float32)
    W3p = jnp.concatenate([W3, jnp.zeros((_H, _H - _DO), jnp.float32)], axis=1)
    b3p = jnp.concatenate([b3, jnp.zeros((_H - _DO,), jnp.float32)])

    h1 = _tc_h1(x, W1)
    degp = _sc_degree(dstp, ones16, z16)
    d, g1 = _tc_prep(degp, h1)
    p1 = _sc_scatter(g1, srcp, dstp, z128, _H)
    g2 = _tc_layer(p1, g1, d, b1.reshape(1, _H), W2, _H)
    p2 = _sc_scatter(g2, srcp, dstp, z128, _H)
    g3 = _tc_layer(p2, g2, d, b2.reshape(1, _H), W3p, _H)
    p3 = _sc_scatter(g3, srcp, dstp, z128, _H)
    return _tc_final(p3, g3, d, b3p.reshape(1, _H))
